# baseline (speedup 1.0000x reference)
import functools

import jax
import jax.numpy as jnp
from jax import lax
from jax.experimental import pallas as pl
from jax.experimental.pallas import tpu as pltpu
from jax.experimental.pallas import tpu_sc as plsc

NUM_SPEAKERS = 100000
EMBED_DIM = 64
BATCH = 16384

NUM_CORES = 2
NUM_SUBCORES = 16
NUM_WORKERS = NUM_CORES * NUM_SUBCORES
B_PER_W = BATCH // NUM_WORKERS
LANES = 16


def _make_gather():
    mesh = plsc.VectorSubcoreMesh(
        core_axis_name="c", subcore_axis_name="s"
    )

    @functools.partial(
        pl.kernel,
        mesh=mesh,
        out_type=jax.ShapeDtypeStruct((EMBED_DIM, BATCH), jnp.float32),
        scratch_types=[
            pltpu.VMEM((B_PER_W,), jnp.int32),
            pltpu.VMEM((B_PER_W, EMBED_DIM), jnp.float32),
            pltpu.VMEM((EMBED_DIM, B_PER_W), jnp.float32),
            pltpu.SemaphoreType.DMA,
        ],
        compiler_params=pltpu.CompilerParams(
            use_tc_tiling_on_sc=False, needs_layout_passes=False
        ),
    )
    def gather(table_hbm, idx_hbm, out_hbm, idx_v, rows_v, rows_t, sem):
        wid = lax.axis_index("s") * NUM_CORES + lax.axis_index("c")
        base = wid * B_PER_W
        pltpu.sync_copy(idx_hbm.at[pl.ds(base, B_PER_W)], idx_v)
        pltpu.async_copy(table_hbm.at[idx_v], rows_v, sem).wait()

        lane = lax.iota(jnp.int32, LANES)

        def transpose_row(d, carry):
            dvec = lane * 0 + d
            for j in range(B_PER_W // LANES):
                bvec = lane + (j * LANES)
                v = plsc.load_gather(rows_v, [bvec, dvec])
                rows_t[d, pl.ds(j * LANES, LANES)] = v
            return carry

        lax.fori_loop(0, EMBED_DIM, transpose_row, 0)
        pltpu.sync_copy(rows_t, out_hbm.at[:, pl.ds(base, B_PER_W)])

    return gather


_gather = _make_gather()


@jax.jit
def kernel(table, spk_id):
    out_t = _gather(table, spk_id.astype(jnp.int32))
    return out_t.T[:, :, None]


# speedup vs baseline: 1.0464x; 1.0464x over previous
import functools

import jax
import jax.numpy as jnp
from jax import lax
from jax.experimental import pallas as pl
from jax.experimental.pallas import tpu as pltpu
from jax.experimental.pallas import tpu_sc as plsc

NUM_SPEAKERS = 100000
EMBED_DIM = 64
BATCH = 16384

NUM_CORES = 2
NUM_SUBCORES = 16
NUM_WORKERS = NUM_CORES * NUM_SUBCORES
B_PER_W = BATCH // NUM_WORKERS
LANES = 16


def _make_gather():
    mesh = plsc.VectorSubcoreMesh(core_axis_name="c", subcore_axis_name="s")

    @functools.partial(
        pl.kernel,
        mesh=mesh,
        out_type=jax.ShapeDtypeStruct((EMBED_DIM * BATCH,), jnp.float32),
        scratch_types=[
            pltpu.VMEM((B_PER_W,), jnp.int32),
            pltpu.VMEM((B_PER_W,), jnp.int32),
            pltpu.VMEM((B_PER_W,), jnp.int32),
            pltpu.VMEM((2 * B_PER_W,), jnp.float32),
            pltpu.SemaphoreType.DMA,
            pltpu.SemaphoreType.DMA,
            pltpu.SemaphoreType.DMA,
            pltpu.SemaphoreType.DMA,
        ],
        compiler_params=pltpu.CompilerParams(
            use_tc_tiling_on_sc=False, needs_layout_passes=False
        ),
    )
    def gather(tflat_hbm, idx_hbm, out_hbm, idx_v, ia_v, ib_v, rows_v,
               sga, sgb, swa, swb):
        wid = lax.axis_index("s") * NUM_CORES + lax.axis_index("c")
        base = wid * B_PER_W
        pltpu.sync_copy(idx_hbm.at[pl.ds(base, B_PER_W)], idx_v)

        def do_pair(d2, carry):
            d0 = d2 * 2
            for j in range(B_PER_W // LANES):
                iv = idx_v[pl.ds(j * LANES, LANES)]
                ia_v[pl.ds(j * LANES, LANES)] = iv + d0 * NUM_SPEAKERS
            ca = pltpu.async_copy(
                tflat_hbm.at[ia_v], rows_v.at[pl.ds(0, B_PER_W)], sga
            )
            for j in range(B_PER_W // LANES):
                iv = idx_v[pl.ds(j * LANES, LANES)]
                ib_v[pl.ds(j * LANES, LANES)] = iv + (d0 + 1) * NUM_SPEAKERS
            cb = pltpu.async_copy(
                tflat_hbm.at[ib_v], rows_v.at[pl.ds(B_PER_W, B_PER_W)], sgb
            )
            ca.wait()
            wa = pltpu.async_copy(
                rows_v.at[pl.ds(0, B_PER_W)],
                out_hbm.at[pl.ds(d0 * BATCH + base, B_PER_W)],
                swa,
            )
            cb.wait()
            wb = pltpu.async_copy(
                rows_v.at[pl.ds(B_PER_W, B_PER_W)],
                out_hbm.at[pl.ds((d0 + 1) * BATCH + base, B_PER_W)],
                swb,
            )
            wa.wait()
            wb.wait()
            return carry

        lax.fori_loop(0, EMBED_DIM // 2, do_pair, 0)

    return gather


_gather = _make_gather()


@jax.jit
def kernel(table, spk_id):
    tflat = table.T.reshape(-1)
    out_flat = _gather(tflat, spk_id.astype(jnp.int32))
    return out_flat.reshape(EMBED_DIM, BATCH).T[:, :, None]


# speedup vs baseline: 1.1453x; 1.0945x over previous
import functools

import jax
import jax.numpy as jnp
from jax import lax
from jax.experimental import pallas as pl
from jax.experimental.pallas import tpu as pltpu
from jax.experimental.pallas import tpu_sc as plsc

NUM_SPEAKERS = 100000
EMBED_DIM = 64
BATCH = 16384

NUM_CORES = 2
NUM_SUBCORES = 16
NUM_WORKERS = NUM_CORES * NUM_SUBCORES
B_PER_W = BATCH // NUM_WORKERS
LANES = 16
DEPTH = 4


def _make_gather():
    mesh = plsc.VectorSubcoreMesh(core_axis_name="c", subcore_axis_name="s")

    @functools.partial(
        pl.kernel,
        mesh=mesh,
        out_type=jax.ShapeDtypeStruct((EMBED_DIM, BATCH), jnp.float32),
        scratch_types=[
            pltpu.VMEM((B_PER_W,), jnp.int32),
            pltpu.VMEM((DEPTH * B_PER_W,), jnp.float32),
            [pltpu.SemaphoreType.DMA] * DEPTH,
            [pltpu.SemaphoreType.DMA] * DEPTH,
        ],
        compiler_params=pltpu.CompilerParams(
            use_tc_tiling_on_sc=False, needs_layout_passes=False
        ),
    )
    def gather(t2d_hbm, idx_hbm, out_hbm, idx_v, rows_v, gsems, wsems):
        wid = lax.axis_index("s") * NUM_CORES + lax.axis_index("c")
        base = wid * B_PER_W
        pltpu.sync_copy(idx_hbm.at[pl.ds(base, B_PER_W)], idx_v)

        def slot(r):
            return pl.ds(r * B_PER_W, B_PER_W)

        def fire(d, r):
            return pltpu.async_copy(
                t2d_hbm.at[d].at[idx_v], rows_v.at[slot(r)], gsems[r]
            )

        def drain(d, r):
            pltpu.make_async_copy(
                t2d_hbm.at[d].at[idx_v], rows_v.at[slot(r)], gsems[r]
            ).wait()
            return pltpu.async_copy(
                rows_v.at[slot(r)],
                out_hbm.at[d, pl.ds(base, B_PER_W)],
                wsems[r],
            )

        def wait_wb(d, r):
            pltpu.make_async_copy(
                rows_v.at[slot(r)],
                out_hbm.at[d, pl.ds(base, B_PER_W)],
                wsems[r],
            ).wait()

        for r in range(DEPTH):
            fire(r, r)

        def body(k, carry):
            d0 = k * DEPTH
            for r in range(DEPTH):
                drain(d0 + r, r)
            for r in range(DEPTH):
                @pl.when(k > 0)
                def _():
                    wait_wb(d0 - DEPTH + r, r)

                @pl.when(k < (EMBED_DIM // DEPTH) - 1)
                def _():
                    fire(d0 + DEPTH + r, r)
            return carry

        lax.fori_loop(0, EMBED_DIM // DEPTH, body, 0)
        for r in range(DEPTH):
            wait_wb(EMBED_DIM - DEPTH + r, r)

    return gather


_gather = _make_gather()


@jax.jit
def kernel(table, spk_id):
    out_t = _gather(table.T, spk_id.astype(jnp.int32))
    return out_t.T[:, :, None]


# speedup vs baseline: 1.3556x; 1.1836x over previous
import functools

import jax
import jax.numpy as jnp
from jax import lax
from jax.experimental import pallas as pl
from jax.experimental.pallas import tpu as pltpu
from jax.experimental.pallas import tpu_sc as plsc

NUM_SPEAKERS = 100000
EMBED_DIM = 64
BATCH = 16384

NUM_CORES = 2
NUM_SUBCORES = 16
NUM_WORKERS = NUM_CORES * NUM_SUBCORES
ROWS_PER_W = EMBED_DIM // NUM_WORKERS
LANES = 16
BCHUNK = 4096
NCHUNK = BATCH // BCHUNK


def _make_gather():
    mesh = plsc.VectorSubcoreMesh(core_axis_name="c", subcore_axis_name="s")

    @functools.partial(
        pl.kernel,
        mesh=mesh,
        out_type=jax.ShapeDtypeStruct((EMBED_DIM, BATCH), jnp.float32),
        scratch_types=[
            pltpu.VMEM((NUM_SPEAKERS,), jnp.float32),
            pltpu.VMEM((2 * BCHUNK,), jnp.int32),
            pltpu.VMEM((2 * BCHUNK,), jnp.float32),
            pltpu.SemaphoreType.DMA,
            [pltpu.SemaphoreType.DMA] * 2,
            [pltpu.SemaphoreType.DMA] * 2,
        ],
        compiler_params=pltpu.CompilerParams(
            use_tc_tiling_on_sc=False, needs_layout_passes=False
        ),
    )
    def gather(t2d_hbm, idx_hbm, out_hbm, row_v, idx_v, outc_v,
               rsem, isems, osems):
        wid = lax.axis_index("s") * NUM_CORES + lax.axis_index("c")

        def slot(r):
            return pl.ds(r * BCHUNK, BCHUNK)

        def fire_idx(c, r):
            return pltpu.async_copy(
                idx_hbm.at[pl.ds(c * BCHUNK, BCHUNK)], idx_v.at[slot(r)],
                isems[r],
            )

        def wait_idx(c, r):
            pltpu.make_async_copy(
                idx_hbm.at[pl.ds(c * BCHUNK, BCHUNK)], idx_v.at[slot(r)],
                isems[r],
            ).wait()

        def wait_out(d, c, r):
            pltpu.make_async_copy(
                outc_v.at[slot(r)],
                out_hbm.at[d, pl.ds(c * BCHUNK, BCHUNK)],
                osems[r],
            ).wait()

        fire_idx(0, 0)
        fire_idx(1, 1)

        def do_row(i, carry):
            d = wid * ROWS_PER_W + i
            pltpu.async_copy(t2d_hbm.at[d], row_v, rsem).wait()

            def do_chunk(c, r):
                @pl.when(c > 1)
                def _():
                    wait_out(d, c - 2, r)

                wait_idx(c, r)

                def gather16(j, carry3):
                    for u in range(4):
                        o = (j * 4 + u) * LANES
                        iv = idx_v[pl.ds(r * BCHUNK + o, LANES)]
                        v = plsc.load_gather(row_v, [iv])
                        outc_v[pl.ds(r * BCHUNK + o, LANES)] = v
                    return carry3

                lax.fori_loop(0, BCHUNK // LANES // 4, gather16, 0)

                pltpu.async_copy(
                    outc_v.at[slot(r)],
                    out_hbm.at[d, pl.ds(c * BCHUNK, BCHUNK)],
                    osems[r],
                )

                nxt = c + 2
                is_last_row = i == ROWS_PER_W - 1

                @pl.when(jnp.logical_or(nxt < NCHUNK,
                                        jnp.logical_not(is_last_row)))
                def _():
                    fire_idx(lax.rem(nxt, NCHUNK), r)

            def do_chunk_pair(kk, carry2):
                do_chunk(kk * 2, 0)
                do_chunk(kk * 2 + 1, 1)
                return carry2

            lax.fori_loop(0, NCHUNK // 2, do_chunk_pair, 0)

            wait_out(d, NCHUNK - 2, 0)
            wait_out(d, NCHUNK - 1, 1)
            return carry

        lax.fori_loop(0, ROWS_PER_W, do_row, 0)

    return gather


_gather = _make_gather()


@jax.jit
def kernel(table, spk_id):
    out_t = _gather(table.T, spk_id.astype(jnp.int32))
    return out_t.T[:, :, None]
